# baseline (speedup 1.0000x reference)
import functools

import jax
import jax.numpy as jnp
from jax import lax
from jax.experimental import pallas as pl
from jax.experimental.pallas import tpu as pltpu

_EPS = 1e-5


def _round_up(x, m):
    return (x + m - 1) // m * m


def _fused_kernel(x_ref, wt_ref, mask_ref, g_ref, b_ref, out_ref, win_ref,
                  yacc_ref, *, k, H, W, Cin, Cout, LW, CW):
    Wp = W + 2
    Ho, Wo = H, W
    PV = Ho * Wp
    x = x_ref[0]

    tail = (PV + (k - 1) * Wp + (k - 1)) // 128 * 128

    @pl.when(pl.program_id(0) == 0)
    def _():
        win_ref[:, tail - 128:] = jnp.zeros((Cin, LW - tail + 128),
                                            jnp.bfloat16)

    for r in range(Ho + 2):
        pr = 1 if r == 0 else (H - 2 if r == Ho + 1 else r - 1)
        row = x[:, pr * W:(pr + 1) * W]
        win_ref[:, r * Wp + 1:r * Wp + 1 + W] = row
        win_ref[:, r * Wp:r * Wp + 1] = row[:, 1:2]
        win_ref[:, r * Wp + 1 + W:r * Wp + 2 + W] = row[:, W - 2:W - 1]

    win = win_ref[...]
    sumv = jnp.zeros((Cout, 1), jnp.float32)
    ssqv = jnp.zeros((Cout, 1), jnp.float32)
    for c in range(PV // CW):
        acc = jnp.zeros((Cout, CW), jnp.float32)
        for tap in range(k * k):
            off = c * CW + (tap // k) * Wp + (tap % k)
            acc = acc + jnp.dot(wt_ref[tap], win[:, off:off + CW],
                                preferred_element_type=jnp.float32)
        yacc_ref[:, c * CW:(c + 1) * CW] = acc.astype(jnp.bfloat16)
        am = acc * mask_ref[:, c * CW:(c + 1) * CW]
        sumv = sumv + jnp.sum(am, axis=-1, keepdims=True)
        ssqv = ssqv + jnp.sum(am * am, axis=-1, keepdims=True)

    mean = sumv / float(Ho * Wo)
    var = jnp.maximum(ssqv / float(Ho * Wo) - mean * mean, 0.0)
    scale = g_ref[...] * lax.rsqrt(var + _EPS)
    shift = b_ref[...] - mean * scale

    for h in range(Ho):
        row = yacc_ref[:, h * Wp:h * Wp + Wo].astype(jnp.float32)
        out_ref[0, :, h * Wo:(h + 1) * Wo] = jnp.maximum(
            row * scale + shift, 0.0)


def kernel(x, weight, bias, gamma, beta):
    del bias
    N, Cin, H, W = x.shape
    Cout = weight.shape[0]
    k = 3

    Wp = W + 2
    Ho, Wo = H, W
    PV = Ho * Wp
    OVR = (k - 1) * Wp + (k - 1)
    LW = _round_up(max(PV + OVR, (Ho + 2) * Wp), 128)
    CW = PV
    for cand in (3, 4, 2, 5):
        if PV % (cand * 128) == 0 and (PV // cand) >= 256:
            CW = PV // cand
            break

    xf = x.reshape(N, Cin, H * W).astype(jnp.bfloat16)

    wt = jnp.transpose(weight, (2, 3, 0, 1)).reshape(k * k, Cout, Cin)
    wt = wt.astype(jnp.bfloat16)

    q = jnp.arange(PV, dtype=jnp.int32)
    mask = ((q % Wp) < Wo).astype(jnp.float32)[None, :]

    g2 = gamma.astype(jnp.float32).reshape(Cout, 1)
    b2 = beta.astype(jnp.float32).reshape(Cout, 1)

    kern = functools.partial(_fused_kernel, k=k, H=H, W=W, Cin=Cin,
                             Cout=Cout, LW=LW, CW=CW)
    y = pl.pallas_call(
        kern,
        out_shape=jax.ShapeDtypeStruct((N, Cout, H * W), jnp.float32),
        grid_spec=pltpu.PrefetchScalarGridSpec(
            num_scalar_prefetch=0,
            grid=(N,),
            in_specs=[
                pl.BlockSpec((1, Cin, H * W), lambda n: (n, 0, 0)),
                pl.BlockSpec((k * k, Cout, Cin), lambda n: (0, 0, 0)),
                pl.BlockSpec((1, PV), lambda n: (0, 0)),
                pl.BlockSpec((Cout, 1), lambda n: (0, 0)),
                pl.BlockSpec((Cout, 1), lambda n: (0, 0)),
            ],
            out_specs=pl.BlockSpec((1, Cout, H * W), lambda n: (n, 0, 0)),
            scratch_shapes=[
                pltpu.VMEM((Cin, LW), jnp.bfloat16),
                pltpu.VMEM((Cout, PV), jnp.bfloat16),
            ],
        ),
        compiler_params=pltpu.CompilerParams(
            dimension_semantics=("parallel",),
            vmem_limit_bytes=48 * 1024 * 1024),
    )(xf, wt, mask, g2, b2)

    return y.reshape(N, Cout, H, W)
